# Initial kernel scaffold; baseline (speedup 1.0000x reference)
import jax
import jax.numpy as jnp
from jax.experimental import pallas as pl


def kernel(outputs, targets):
    raise NotImplementedError("write your pallas kernel here")


# speedup vs baseline: 1.4459x; 1.4459x over previous
import functools

import jax
import jax.numpy as jnp
from jax.experimental import pallas as pl
from jax.experimental.pallas import tpu as pltpu


_GRP = 8


def _body(oxt, oyt, txt, tyt, out_ref, acc_ref):
    n = oxt.shape[0]

    def bf(x):
        return x.astype(jnp.bfloat16).astype(jnp.float32)

    def pass_sum(ax_ref, ay_ref, bx_ref, by_ref):
        ax = ax_ref[...]
        ay = ay_ref[...]
        a2 = ax * ax + ay * ay
        axb = bf(ax)
        ayb = bf(ay)
        acc_ref[...] = jnp.full(acc_ref.shape, 1e30, jnp.float32)

        def grp(g, _):
            base = g * _GRP
            bxg = bx_ref[pl.ds(base, _GRP), :]
            byg = by_ref[pl.ds(base, _GRP), :]
            b2g = bxg * bxg + byg * byg
            bxgb = bf(bxg)
            bygb = bf(byg)
            m = acc_ref[...]
            for k in range(_GRP):
                cross = axb * bxgb[k:k + 1, :] + ayb * bygb[k:k + 1, :]
                s = a2 + b2g[k:k + 1, :]
                m = jnp.minimum(m, s - 2.0 * cross)
            acc_ref[...] = m
            return 0

        jax.lax.fori_loop(0, n // _GRP, grp, 0)
        d2 = jnp.maximum(acc_ref[...], 0.0)
        return jnp.sum(jnp.sqrt(d2 + 1e-12))

    s_ot = pass_sum(oxt, oyt, txt, tyt)
    s_to = pass_sum(txt, tyt, oxt, oyt)
    out_ref[0, 0] = s_ot + s_to


@functools.partial(jax.jit, static_argnames=())
def kernel(outputs, targets):
    bs, f = outputs.shape
    n = f // 2
    oxt = outputs[:, :n].T
    oyt = outputs[:, n:].T
    txt = targets[:, :n].T
    tyt = targets[:, n:].T

    total = pl.pallas_call(
        _body,
        out_shape=jax.ShapeDtypeStruct((1, 1), jnp.float32),
        in_specs=[pl.BlockSpec((n, bs), lambda: (0, 0))] * 4,
        out_specs=pl.BlockSpec(memory_space=pltpu.SMEM),
        scratch_shapes=[pltpu.VMEM((n, bs), jnp.float32)],
    )(oxt, oyt, txt, tyt)

    return total[0, 0] * (0.5 / (bs * n))


# speedup vs baseline: 1.9262x; 1.3321x over previous
import functools

import jax
import jax.numpy as jnp
from jax.experimental import pallas as pl
from jax.experimental.pallas import tpu as pltpu


_GRP = 8


def _body(oxt, oyt, txt, tyt, out_ref, acc_ref):
    n = oxt.shape[0]

    def bf(x):
        return x.astype(jnp.bfloat16).astype(jnp.float32)

    def pass_sum(ax_ref, ay_ref, bx_ref, by_ref):
        ax = ax_ref[...]
        ay = ay_ref[...]
        a2 = ax * ax + ay * ay
        nax = -2.0 * bf(ax)
        nay = -2.0 * bf(ay)
        acc_ref[...] = jnp.full(acc_ref.shape, 1e30, jnp.float32)

        def grp(g, _):
            base = g * _GRP
            bxg = bx_ref[pl.ds(base, _GRP), :]
            byg = by_ref[pl.ds(base, _GRP), :]
            b2g = bxg * bxg + byg * byg
            bxgb = bf(bxg)
            bygb = bf(byg)
            m = acc_ref[...]
            for k in range(_GRP):
                t1 = nax * bxgb[k:k + 1, :] + b2g[k:k + 1, :]
                t2 = nay * bygb[k:k + 1, :] + t1
                m = jnp.minimum(m, t2)
            acc_ref[...] = m
            return 0

        jax.lax.fori_loop(0, n // _GRP, grp, 0)
        d2 = jnp.maximum(acc_ref[...] + a2, 0.0)
        return jnp.sum(jnp.sqrt(d2 + 1e-12))

    s_ot = pass_sum(oxt, oyt, txt, tyt)
    s_to = pass_sum(txt, tyt, oxt, oyt)
    out_ref[0, 0] = s_ot + s_to


@functools.partial(jax.jit, static_argnames=())
def kernel(outputs, targets):
    bs, f = outputs.shape
    n = f // 2
    oxt = outputs[:, :n].T
    oyt = outputs[:, n:].T
    txt = targets[:, :n].T
    tyt = targets[:, n:].T

    total = pl.pallas_call(
        _body,
        out_shape=jax.ShapeDtypeStruct((1, 1), jnp.float32),
        in_specs=[pl.BlockSpec((n, bs), lambda: (0, 0))] * 4,
        out_specs=pl.BlockSpec(memory_space=pltpu.SMEM),
        scratch_shapes=[pltpu.VMEM((n, bs), jnp.float32)],
    )(oxt, oyt, txt, tyt)

    return total[0, 0] * (0.5 / (bs * n))


# speedup vs baseline: 1.9371x; 1.0057x over previous
import functools

import jax
import jax.numpy as jnp
from jax.experimental import pallas as pl
from jax.experimental.pallas import tpu as pltpu


_GRP = 16


def _body(oxt, oyt, txt, tyt, out_ref, acc_ref):
    n = oxt.shape[0]

    def bf(x):
        return x.astype(jnp.bfloat16).astype(jnp.float32)

    def pass_sum(ax_ref, ay_ref, bx_ref, by_ref):
        ax = ax_ref[...]
        ay = ay_ref[...]
        a2 = ax * ax + ay * ay
        nax = -2.0 * bf(ax)
        nay = -2.0 * bf(ay)
        acc_ref[...] = jnp.full(acc_ref.shape, 1e30, jnp.float32)

        def grp(g, _):
            base = g * _GRP
            bxg = bx_ref[pl.ds(base, _GRP), :]
            byg = by_ref[pl.ds(base, _GRP), :]
            b2g = bxg * bxg + byg * byg
            bxgb = bf(bxg)
            bygb = bf(byg)
            m = acc_ref[...]
            for k in range(_GRP):
                t1 = nax * bxgb[k:k + 1, :] + b2g[k:k + 1, :]
                t2 = nay * bygb[k:k + 1, :] + t1
                m = jnp.minimum(m, t2)
            acc_ref[...] = m
            return 0

        jax.lax.fori_loop(0, n // _GRP, grp, 0)
        d2 = jnp.maximum(acc_ref[...] + a2, 0.0)
        return jnp.sum(jnp.sqrt(d2 + 1e-12))

    s_ot = pass_sum(oxt, oyt, txt, tyt)
    s_to = pass_sum(txt, tyt, oxt, oyt)
    out_ref[0, 0] = s_ot + s_to


@functools.partial(jax.jit, static_argnames=())
def kernel(outputs, targets):
    bs, f = outputs.shape
    n = f // 2
    oxt = outputs[:, :n].T
    oyt = outputs[:, n:].T
    txt = targets[:, :n].T
    tyt = targets[:, n:].T

    total = pl.pallas_call(
        _body,
        out_shape=jax.ShapeDtypeStruct((1, 1), jnp.float32),
        in_specs=[pl.BlockSpec((n, bs), lambda: (0, 0))] * 4,
        out_specs=pl.BlockSpec(memory_space=pltpu.SMEM),
        scratch_shapes=[pltpu.VMEM((n, bs), jnp.float32)],
    )(oxt, oyt, txt, tyt)

    return total[0, 0] * (0.5 / (bs * n))


# speedup vs baseline: 2.0677x; 1.0674x over previous
import functools

import jax
import jax.numpy as jnp
from jax.experimental import pallas as pl
from jax.experimental.pallas import tpu as pltpu


_GRP = 16


def _body(outs, tgts, out_ref, oxt, oyt, txt, tyt, acc_ref):
    bs = outs.shape[0]
    n = outs.shape[1] // 2

    oxt[...] = outs[:, :n].T
    oyt[...] = outs[:, n:].T
    txt[...] = tgts[:, :n].T
    tyt[...] = tgts[:, n:].T

    def bf(x):
        return x.astype(jnp.bfloat16).astype(jnp.float32)

    def pass_sum(ax_ref, ay_ref, bx_ref, by_ref):
        ax = ax_ref[...]
        ay = ay_ref[...]
        a2 = ax * ax + ay * ay
        nax = -2.0 * bf(ax)
        nay = -2.0 * bf(ay)
        acc_ref[...] = jnp.full(acc_ref.shape, 1e30, jnp.float32)

        def grp(g, _):
            base = g * _GRP
            bxg = bx_ref[pl.ds(base, _GRP), :]
            byg = by_ref[pl.ds(base, _GRP), :]
            b2g = bxg * bxg + byg * byg
            bxgb = bf(bxg)
            bygb = bf(byg)
            m = acc_ref[...]
            for k in range(_GRP):
                t1 = nax * bxgb[k:k + 1, :] + b2g[k:k + 1, :]
                t2 = nay * bygb[k:k + 1, :] + t1
                m = jnp.minimum(m, t2)
            acc_ref[...] = m
            return 0

        jax.lax.fori_loop(0, n // _GRP, grp, 0)
        d2 = jnp.maximum(acc_ref[...] + a2, 0.0)
        return jnp.sum(jnp.sqrt(d2 + 1e-12))

    s_ot = pass_sum(oxt, oyt, txt, tyt)
    s_to = pass_sum(txt, tyt, oxt, oyt)
    out_ref[0, 0] = s_ot + s_to


@functools.partial(jax.jit, static_argnames=())
def kernel(outputs, targets):
    bs, f = outputs.shape
    n = f // 2

    total = pl.pallas_call(
        _body,
        out_shape=jax.ShapeDtypeStruct((1, 1), jnp.float32),
        in_specs=[pl.BlockSpec((bs, f), lambda: (0, 0))] * 2,
        out_specs=pl.BlockSpec(memory_space=pltpu.SMEM),
        scratch_shapes=[pltpu.VMEM((n, bs), jnp.float32)] * 5,
    )(outputs, targets)

    return total[0, 0] * (0.5 / (bs * n))
